# Initial kernel scaffold; baseline (speedup 1.0000x reference)
import jax
import jax.numpy as jnp
from jax.experimental import pallas as pl


def kernel(x, edge_index, eps, W1a, b1a, W1b, b1b, W2a, b2a, W2b, b2b):
    raise NotImplementedError("write your pallas kernel here")


# speedup vs baseline: 7.0885x; 7.0885x over previous
import functools

import jax
import jax.numpy as jnp
from jax import lax
from jax.experimental import pallas as pl
from jax.experimental.pallas import tpu as pltpu
from jax.experimental.pallas import tpu_sc as plsc

_CHUNK = 80


@functools.cache
def _make_spmm(N, D, E):
    info = plsc.get_sparse_core_info()
    NC, NS = info.num_cores, info.num_subcores
    NW = NC * NS
    per_t = E // NW
    n_chunks = per_t // _CHUNK
    ZROWS = 8
    N_pad = -(-N // (ZROWS * NS)) * (ZROWS * NS)
    rows_per_tile = N_pad // NS
    n_z = rows_per_tile // ZROWS

    mesh = plsc.VectorSubcoreMesh(core_axis_name="c", subcore_axis_name="s")

    @functools.partial(
        pl.kernel,
        mesh=mesh,
        out_type=jax.ShapeDtypeStruct((NC, N_pad, D), jnp.float32),
        scratch_types=[
            pltpu.VMEM((n_chunks, _CHUNK), jnp.int32),
            pltpu.VMEM((n_chunks, _CHUNK), jnp.int32),
            pltpu.VMEM((_CHUNK, D), jnp.float32),
            pltpu.VMEM((ZROWS, D), jnp.float32),
            pltpu.VMEM_SHARED((N_pad, D), jnp.float32),
            pltpu.SemaphoreType.DMA,
        ],
    )
    def spmm(x_hbm, src_hbm, dst_hbm, out_hbm, sidx, didx, rows, zbuf, acc, sem):
        cid = lax.axis_index("c")
        sid = lax.axis_index("s")
        wid = sid * NC + cid

        for i in range(ZROWS):
            for j in range(D // 16):
                zbuf[i, pl.ds(j * 16, 16)] = jnp.zeros((16,), jnp.float32)

        def zacc(k, carry):
            pltpu.sync_copy(zbuf, acc.at[pl.ds(sid * rows_per_tile + k * ZROWS, ZROWS)])
            return carry

        lax.fori_loop(0, n_z, zacc, 0)

        pltpu.sync_copy(src_hbm.at[wid], sidx)
        pltpu.sync_copy(dst_hbm.at[wid], didx)
        plsc.subcore_barrier()

        def chunk(ci, carry):
            pltpu.async_copy(x_hbm.at[sidx.at[ci]], rows, sem).wait()
            pltpu.sync_copy(rows, acc.at[didx.at[ci]], add=True)
            return carry

        lax.fori_loop(0, n_chunks, chunk, 0)
        plsc.subcore_barrier()

        base = sid * rows_per_tile
        pltpu.sync_copy(acc.at[pl.ds(base, rows_per_tile)],
                        out_hbm.at[cid, pl.ds(base, rows_per_tile)])

    return spmm


@functools.cache
def _make_mlp(N, D, BLK=1000):
    def body(eps_ref, x_ref, p0_ref, p1_ref, wa_ref, ba_ref, wb_ref, bb_ref, o_ref):
        scale = 1.0 + eps_ref[0]
        hin = x_ref[:] * scale + p0_ref[:] + p1_ref[:]
        t = lax.dot_general(hin, wa_ref[:], (((1,), (1,)), ((), ())),
                            preferred_element_type=jnp.float32)
        t = jnp.maximum(t + ba_ref[:], 0.0)
        o = lax.dot_general(t, wb_ref[:], (((1,), (1,)), ((), ())),
                            preferred_element_type=jnp.float32)
        o_ref[:] = o + bb_ref[:]

    return pl.pallas_call(
        body,
        grid=(N // BLK,),
        in_specs=[
            pl.BlockSpec(memory_space=pltpu.SMEM),
            pl.BlockSpec((BLK, D), lambda i: (i, 0)),
            pl.BlockSpec((BLK, D), lambda i: (i, 0)),
            pl.BlockSpec((BLK, D), lambda i: (i, 0)),
            pl.BlockSpec((D, D), lambda i: (0, 0)),
            pl.BlockSpec((1, D), lambda i: (0, 0)),
            pl.BlockSpec((D, D), lambda i: (0, 0)),
            pl.BlockSpec((1, D), lambda i: (0, 0)),
        ],
        out_specs=pl.BlockSpec((BLK, D), lambda i: (i, 0)),
        out_shape=jax.ShapeDtypeStruct((N, D), jnp.float32),
    )


def kernel(x, edge_index, eps, W1a, b1a, W1b, b1b, W2a, b2a, W2b, b2b):
    N, D = x.shape
    E = edge_index.shape[1]
    info = plsc.get_sparse_core_info()
    NW = info.num_cores * info.num_subcores
    src = edge_index[0].astype(jnp.int32).reshape(NW, E // (NW * _CHUNK), _CHUNK)
    dst = edge_index[1].astype(jnp.int32).reshape(NW, E // (NW * _CHUNK), _CHUNK)
    eps1 = jnp.asarray(eps, jnp.float32).reshape(1)

    spmm = _make_spmm(N, D, E)
    mlp = _make_mlp(N, D)

    p = spmm(x, src, dst)
    h = mlp(eps1, x, p[0], p[1], W1a, b1a.reshape(1, D), W1b, b1b.reshape(1, D))
    p2 = spmm(h, src, dst)
    out = mlp(eps1, h, p2[0], p2[1], W2a, b2a.reshape(1, D), W2b, b2b.reshape(1, D))
    return out
